# Initial kernel scaffold; baseline (speedup 1.0000x reference)
import functools

import jax
import jax.numpy as jnp
from jax import lax
from jax.experimental import pallas as pl
from jax.experimental.pallas import tpu as pltpu
from jax.experimental.pallas import tpu_sc as plsc

D_MODEL = 64
NUM_CORES = 2
NUM_SUBCORES = 16
NW = NUM_CORES * NUM_SUBCORES
CHUNK = 128
NBUF = 8


@functools.partial(jax.jit, static_argnames=("total", "n_chunks"))
def _emb_lookup(table, idx3, total, n_chunks):
    mesh = plsc.VectorSubcoreMesh(
        core_axis_name="c", subcore_axis_name="s",
        num_cores=NUM_CORES, num_subcores=NUM_SUBCORES)
    b_per_w = n_chunks * CHUNK

    @functools.partial(
        pl.kernel,
        out_type=jax.ShapeDtypeStruct((total, D_MODEL), jnp.float32),
        mesh=mesh,
        scratch_types=[
            pltpu.VMEM((n_chunks, CHUNK), jnp.int32),
            pltpu.VMEM((NBUF, CHUNK, D_MODEL), jnp.float32),
            pltpu.SemaphoreType.DMA,
            pltpu.SemaphoreType.DMA((NBUF,)),
            pltpu.SemaphoreType.DMA((NBUF,)),
        ],
    )
    def k(table_hbm, idx_hbm, out_hbm, idx_v, rows_v, isem, gsems, osems):
        wid = lax.axis_index("s") * NUM_CORES + lax.axis_index("c")
        base = wid * b_per_w

        cp = pltpu.make_async_copy(idx_hbm.at[wid], idx_v, isem)
        cp.start()
        cp.wait()

        def g_copy(j, b):
            return pltpu.make_async_copy(
                table_hbm.at[idx_v.at[j]], rows_v.at[b], gsems.at[b])

        def o_copy(j, b):
            return pltpu.make_async_copy(
                rows_v.at[b],
                out_hbm.at[pl.ds(base + j * CHUNK, CHUNK)],
                osems.at[b])

        for b in range(NBUF):
            g_copy(b, b).start()

        n_rounds = n_chunks // NBUF

        def round_body(r, carry):
            for b in range(NBUF):
                j = r * NBUF + b
                g_copy(j, b).wait()
                o_copy(j, b).start()
            for b in range(NBUF):
                j = r * NBUF + b
                o_copy(j, b).wait()
                jn = j + NBUF

                @pl.when(jn < n_chunks)
                def _():
                    g_copy(jn, b).start()

            return carry

        lax.fori_loop(0, n_rounds, round_body, 0)

    return k(table, idx3)


def kernel(x, table):
    bsz, seq = x.shape
    total = bsz * seq
    n_chunks = total // (NW * CHUNK)
    idx3 = x.reshape(NW, n_chunks, CHUNK)
    out = _emb_lookup(table, idx3, total, n_chunks)
    return out.reshape(bsz, seq, D_MODEL)


# speedup vs baseline: 4.2537x; 4.2537x over previous
import functools

import jax
import jax.numpy as jnp
from jax import lax
from jax.experimental import pallas as pl
from jax.experimental.pallas import tpu as pltpu
from jax.experimental.pallas import tpu_sc as plsc

D_MODEL = 64
NUM_CORES = 2
NUM_SUBCORES = 16
NW = NUM_CORES * NUM_SUBCORES
CHUNK = 128
NBUF = 8


@functools.partial(jax.jit, static_argnames=("total", "n_chunks"))
def _emb_lookup(table, idx3, total, n_chunks):
    mesh = plsc.VectorSubcoreMesh(
        core_axis_name="c", subcore_axis_name="s",
        num_cores=NUM_CORES, num_subcores=NUM_SUBCORES)
    b_per_w = n_chunks * CHUNK

    @functools.partial(
        pl.kernel,
        out_type=jax.ShapeDtypeStruct((total, D_MODEL), jnp.float32),
        mesh=mesh,
        scratch_types=[
            pltpu.VMEM((n_chunks, CHUNK), jnp.int32),
            pltpu.VMEM((NBUF, CHUNK, D_MODEL), jnp.float32),
            pltpu.SemaphoreType.DMA,
            pltpu.SemaphoreType.DMA((NBUF,)),
            pltpu.SemaphoreType.DMA((NBUF,)),
        ],
        compiler_params=pltpu.CompilerParams(use_tc_tiling_on_sc=False),
    )
    def k(table_hbm, idx_hbm, out_hbm, idx_v, rows_v, isem, gsems, osems):
        wid = lax.axis_index("s") * NUM_CORES + lax.axis_index("c")
        base = wid * b_per_w

        cp = pltpu.make_async_copy(idx_hbm.at[wid], idx_v, isem)
        cp.start()
        cp.wait()

        def g_copy(j, b):
            return pltpu.make_async_copy(
                table_hbm.at[idx_v.at[j]], rows_v.at[b], gsems.at[b])

        def o_copy(j, b):
            return pltpu.make_async_copy(
                rows_v.at[b],
                out_hbm.at[pl.ds(base + j * CHUNK, CHUNK)],
                osems.at[b])

        for b in range(NBUF):
            g_copy(b, b).start()

        n_rounds = n_chunks // NBUF

        def round_body(r, carry):
            for b in range(NBUF):
                j = r * NBUF + b
                g_copy(j, b).wait()
                o_copy(j, b).start()
            for b in range(NBUF):
                j = r * NBUF + b
                o_copy(j, b).wait()
                jn = j + NBUF

                @pl.when(jn < n_chunks)
                def _():
                    g_copy(jn, b).start()

            return carry

        lax.fori_loop(0, n_rounds, round_body, 0)

    return k(table, idx3)


def kernel(x, table):
    bsz, seq = x.shape
    total = bsz * seq
    n_chunks = total // (NW * CHUNK)
    idx3 = x.reshape(NW, n_chunks, CHUNK)
    out = _emb_lookup(table, idx3, total, n_chunks)
    return out.reshape(bsz, seq, D_MODEL)
